# baseline (speedup 1.0000x reference)
import functools

import jax
import jax.numpy as jnp
from jax import lax
from jax.experimental import pallas as pl
from jax.experimental.pallas import tpu as pltpu
from jax.experimental.pallas import tpu_sc as plsc

_B = 16384
_FIELD0 = 1000000
_D = 16
_ROWS = 2 * _FIELD0
_NC, _NS = 2, 16
_NW = _NC * _NS
_SPW = _B // _NW
_CW = 128
_NCH = _SPW // _CW
_LANE_TILES = _ROWS // 128
_PLANE = 8 * 128 * _LANE_TILES


def _gather_body(flat_hbm, xv_hbm, out_hbm, x_v, idx_v, emb_v, sem):
    wid = lax.axis_index("s") * _NC + lax.axis_index("c")
    base = wid * _SPW
    pltpu.sync_copy(xv_hbm.at[pl.ds(8 * wid, 8)], x_v)

    def build_idx(c, carry):
        for f in range(2):
            for k in range(8):
                g = x_v[2 * c + f, pl.ds(k * 16, 16)] + (f * _FIELD0)
                bvec = lax.shift_right_logical(g, 7) * 1024 + \
                    lax.bitwise_and(g, 127)
                for d in range(_D):
                    dd = f * _D + d
                    idx_v[dd, pl.ds(c * _CW + k * 16, 16)] = (
                        bvec + ((d // 8) * _PLANE + (d % 8) * 128))
        return carry

    def per_dim(c):
        def fire(dd, carry):
            pltpu.make_async_copy(
                flat_hbm.at[idx_v.at[dd, pl.ds(c * _CW, _CW)]],
                emb_v.at[dd // 8, c, dd % 8, :], sem).start()
            return carry
        lax.fori_loop(0, 2 * _D, fire, 0)

    for c in range(_NCH):
        build_idx(c, 0)
        per_dim(c)
    dst = out_hbm.at[:, pl.ds(_NCH * wid, _NCH), :, :]
    pltpu.make_async_copy(dst, emb_v, sem).wait()
    pltpu.sync_copy(emb_v, dst)


@functools.cache
def _make_gather():
    return pl.kernel(
        _gather_body,
        out_type=jax.ShapeDtypeStruct((4, _B // _CW, 8, _CW), jnp.float32),
        mesh=plsc.VectorSubcoreMesh(core_axis_name="c", subcore_axis_name="s",
                                    num_cores=_NC, num_subcores=_NS),
        scratch_types=[
            pltpu.VMEM((8, _CW), jnp.int32),
            pltpu.VMEM((2 * _D, _SPW), jnp.int32),
            pltpu.VMEM((4, _NCH, 8, _CW), jnp.float32),
            pltpu.SemaphoreType.DMA,
        ],
        compiler_params=pltpu.CompilerParams(use_tc_tiling_on_sc=False),
    )


def _bn_relu_t(h, g, bt):
    m = jnp.mean(h, axis=1, keepdims=True)
    v = jnp.mean((h - m) ** 2, axis=1, keepdims=True)
    return jnp.maximum((h - m) * lax.rsqrt(v + 1e-5) * g[:, None] + bt[:, None],
                       0.0)


def _mlp_body(embt_ref, w1_ref, b1_ref, g1_ref, bt1_ref, w2_ref, b2_ref, g2_ref,
              bt2_ref, w3_ref, b3_ref, g3_ref, bt3_ref, wfc_ref, bfc_ref,
              out_ref):
    e = embt_ref[...]
    h = jnp.dot(w1_ref[...], e, preferred_element_type=jnp.float32)
    h = _bn_relu_t(h + b1_ref[...][:, None], g1_ref[...], bt1_ref[...])
    h = jnp.dot(w2_ref[...], h, preferred_element_type=jnp.float32)
    h = _bn_relu_t(h + b2_ref[...][:, None], g2_ref[...], bt2_ref[...])
    h = jnp.dot(w3_ref[...], h, preferred_element_type=jnp.float32)
    h = _bn_relu_t(h + b3_ref[...][:, None], g3_ref[...], bt3_ref[...])
    gmf = e[:_D] * e[_D:]
    z = jnp.concatenate([gmf, h], axis=0)
    out_ref[...] = jnp.sum(z * wfc_ref[...][:, None], axis=0) + bfc_ref[0]


_mlp = pl.pallas_call(
    _mlp_body,
    out_shape=jax.ShapeDtypeStruct((_B,), jnp.float32),
)


def kernel(x, table, W1, b1, g1, bt1, W2, b2, g2, bt2, W3, b3, g3, bt3, Wfc, bfc):
    xv = (x.astype(jnp.int32).reshape(128, 128, 2)
          .transpose(0, 2, 1).reshape(256, 128))
    flat = (table.T.reshape(2, 8, _LANE_TILES, 128)
            .transpose(0, 2, 1, 3).reshape(-1))
    embt4 = _make_gather()(flat, xv)
    embt = embt4.transpose(0, 2, 1, 3).reshape(2 * _D, _B)
    return _mlp(embt, W1, b1, g1, bt1, W2, b2, g2, bt2, W3, b3, g3, bt3,
                Wfc[0], bfc)


# speedup vs baseline: 1.0161x; 1.0161x over previous
import functools

import jax
import jax.numpy as jnp
from jax import lax
from jax.experimental import pallas as pl
from jax.experimental.pallas import tpu as pltpu
from jax.experimental.pallas import tpu_sc as plsc

_B = 16384
_FIELD0 = 1000000
_D = 16
_ROWS = 2 * _FIELD0
_NC, _NS = 2, 16
_NW = _NC * _NS
_SPW = _B // _NW
_CW = 128
_NCH = _SPW // _CW
_LANE_TILES = _ROWS // 128
_PLANE = 8 * 128 * _LANE_TILES


def _gather_body(flat_hbm, xv_hbm, out_hbm, x_v, idx_v, emb_v, sem):
    wid = lax.axis_index("s") * _NC + lax.axis_index("c")
    base = wid * _SPW
    pltpu.sync_copy(xv_hbm.at[pl.ds(8 * wid, 8)], x_v)

    def build_idx(c, carry):
        for f in range(2):
            for k in range(8):
                g = x_v[2 * c + f, pl.ds(k * 16, 16)] + (f * _FIELD0)
                bvec = lax.shift_right_logical(g, 7) * 1024 + \
                    lax.bitwise_and(g, 127)
                for d in range(_D):
                    dd = f * _D + d
                    idx_v[dd, pl.ds(c * _CW + k * 16, 16)] = (
                        bvec + ((d // 8) * _PLANE + (d % 8) * 128))
        return carry

    lax.fori_loop(0, _NCH, build_idx, 0)

    def per_dim(dd, carry):
        for c in range(_NCH):
            pltpu.make_async_copy(
                flat_hbm.at[idx_v.at[dd, pl.ds(c * _CW, _CW)]],
                emb_v.at[dd // 8, c, dd % 8, :], sem).start()
        return carry

    lax.fori_loop(0, 2 * _D, per_dim, 0)
    dst = out_hbm.at[:, pl.ds(_NCH * wid, _NCH), :, :]
    pltpu.make_async_copy(dst, emb_v, sem).wait()
    pltpu.sync_copy(emb_v, dst)


@functools.cache
def _make_gather():
    return pl.kernel(
        _gather_body,
        out_type=jax.ShapeDtypeStruct((4, _B // _CW, 8, _CW), jnp.float32),
        mesh=plsc.VectorSubcoreMesh(core_axis_name="c", subcore_axis_name="s",
                                    num_cores=_NC, num_subcores=_NS),
        scratch_types=[
            pltpu.VMEM((8, _CW), jnp.int32),
            pltpu.VMEM((2 * _D, _SPW), jnp.int32),
            pltpu.VMEM((4, _NCH, 8, _CW), jnp.float32),
            pltpu.SemaphoreType.DMA,
        ],
        compiler_params=pltpu.CompilerParams(use_tc_tiling_on_sc=False),
    )


def _bn_relu_t(h, g, bt):
    m = jnp.mean(h, axis=1, keepdims=True)
    v = jnp.mean((h - m) ** 2, axis=1, keepdims=True)
    return jnp.maximum((h - m) * lax.rsqrt(v + 1e-5) * g[:, None] + bt[:, None],
                       0.0)


def _mlp_body(embt_ref, w1_ref, b1_ref, g1_ref, bt1_ref, w2_ref, b2_ref, g2_ref,
              bt2_ref, w3_ref, b3_ref, g3_ref, bt3_ref, wfc_ref, bfc_ref,
              out_ref):
    e = embt_ref[...]
    h = jnp.dot(w1_ref[...], e, preferred_element_type=jnp.float32)
    h = _bn_relu_t(h + b1_ref[...][:, None], g1_ref[...], bt1_ref[...])
    h = jnp.dot(w2_ref[...], h, preferred_element_type=jnp.float32)
    h = _bn_relu_t(h + b2_ref[...][:, None], g2_ref[...], bt2_ref[...])
    h = jnp.dot(w3_ref[...], h, preferred_element_type=jnp.float32)
    h = _bn_relu_t(h + b3_ref[...][:, None], g3_ref[...], bt3_ref[...])
    gmf = e[:_D] * e[_D:]
    z = jnp.concatenate([gmf, h], axis=0)
    out_ref[...] = jnp.sum(z * wfc_ref[...][:, None], axis=0) + bfc_ref[0]


_mlp = pl.pallas_call(
    _mlp_body,
    out_shape=jax.ShapeDtypeStruct((_B,), jnp.float32),
)


def kernel(x, table, W1, b1, g1, bt1, W2, b2, g2, bt2, W3, b3, g3, bt3, Wfc, bfc):
    xv = (x.astype(jnp.int32).reshape(128, 128, 2)
          .transpose(0, 2, 1).reshape(256, 128))
    flat = (table.T.reshape(2, 8, _LANE_TILES, 128)
            .transpose(0, 2, 1, 3).reshape(-1))
    embt4 = _make_gather()(flat, xv)
    embt = embt4.transpose(0, 2, 1, 3).reshape(2 * _D, _B)
    return _mlp(embt, W1, b1, g1, bt1, W2, b2, g2, bt2, W3, b3, g3, bt3,
                Wfc[0], bfc)
